# baseline (speedup 1.0000x reference)
import functools

import jax
import jax.numpy as jnp
from jax import lax
from jax.experimental import pallas as pl
from jax.experimental.pallas import tpu as pltpu
from jax.experimental.pallas import tpu_sc as plsc

_NCLS_PAD = 128
_MLP_BLOCK = 512
_LANES = 16
_NBUF = 8


@functools.lru_cache(maxsize=None)
def _make_pooled_gather(V, D, B, S):
    info = plsc.get_sparse_core_info()
    NC, NS = info.num_cores, info.num_subcores
    NW = NC * NS
    assert B % NW == 0 and (B // NW) % _LANES == 0
    b_per_w = B // NW
    row0 = ((S - 1) // 8) * 8
    sub = (S - 1) % 8
    mesh = plsc.VectorSubcoreMesh(core_axis_name="c", subcore_axis_name="s")

    @functools.partial(
        pl.kernel,
        mesh=mesh,
        compiler_params=pltpu.CompilerParams(needs_layout_passes=False),
        out_type=jax.ShapeDtypeStruct((B, 2 * D), jnp.float32),
        scratch_types=[
            pltpu.VMEM((8, b_per_w), jnp.int32),
            pltpu.VMEM((b_per_w + 2 * _LANES,), jnp.int32),
            pltpu.VMEM((b_per_w,), jnp.int32),
            pltpu.VMEM((_NBUF, D, 128), jnp.float32),
            pltpu.VMEM((b_per_w, 2 * D), jnp.float32),
        ] + [pltpu.SemaphoreType.DMA] * _NBUF,
    )
    def gather_last(idsT_hbm, tabT_hbm, out_hbm, ids_v, col_v, lane_v,
                    slabs_v, out_v, *sems):
        wid = lax.axis_index("s") * NC + lax.axis_index("c")
        base = wid * b_per_w
        pltpu.sync_copy(idsT_hbm.at[pl.ds(row0, 8), pl.ds(base, b_per_w)],
                        ids_v)
        for t in range(b_per_w // _LANES):
            ids16 = ids_v[sub, pl.ds(t * _LANES, _LANES)]
            col_v[pl.ds(t * _LANES, _LANES)] = lax.bitwise_and(ids16, ~127)
            lane_v[pl.ds(t * _LANES, _LANES)] = lax.bitwise_and(ids16, 127)

        def fire_slab(c0, slot):
            pltpu.async_copy(tabT_hbm.at[pl.ds(0, D // 2), pl.ds(c0, 128)],
                             slabs_v.at[slot, pl.ds(0, D // 2)], sems[slot])
            pltpu.async_copy(tabT_hbm.at[pl.ds(D // 2, D // 2), pl.ds(c0, 128)],
                             slabs_v.at[slot, pl.ds(D // 2, D // 2)],
                             sems[slot])

        head = col_v[pl.ds(0, _LANES)]
        for j in range(_NBUF):
            c0 = pl.multiple_of(head[j], 128)
            fire_slab(c0, j)

        def body(t, carry):
            col16 = col_v[pl.ds(t * _LANES, _LANES)]
            coln = col_v[pl.ds(t * _LANES + _LANES, _LANES)]
            lane16 = lane_v[pl.ds(t * _LANES, _LANES)]
            for j2 in range(_LANES):
                j = t * _LANES + j2
                slot = j2 % _NBUF
                for _h in range(2):
                    pltpu.make_async_copy(
                        tabT_hbm.at[pl.ds(0, D // 2), pl.ds(0, 128)],
                        slabs_v.at[slot, pl.ds(0, D // 2)], sems[slot]).wait()
                ln = jnp.broadcast_to(lane16[j2], (_LANES,))
                for k in range(D // _LANES):
                    e16 = k * _LANES + lax.iota(jnp.int32, _LANES)
                    v = plsc.load_gather(slabs_v.at[slot], [e16, ln])
                    out_v[j, pl.ds(k * _LANES, _LANES)] = v
                    out_v[j, pl.ds(D + k * _LANES, _LANES)] = v
                c0n = coln[j2 - _NBUF] if j2 >= _NBUF else col16[j2 + _NBUF]

                @pl.when(j + _NBUF < b_per_w)
                def _():
                    c0a = pl.multiple_of(c0n, 128)
                    fire_slab(c0a, slot)

            return carry

        lax.fori_loop(0, b_per_w // _LANES, body, 0)
        pltpu.sync_copy(out_v, out_hbm.at[pl.ds(base, b_per_w)])

    return gather_last


def _mlp_body(x_ref, w1_ref, b1_ref, w2_ref, b2_ref, out_ref):
    h = jnp.dot(x_ref[...], w1_ref[...], preferred_element_type=jnp.float32)
    h = jnp.maximum(h + b1_ref[...], 0.0)
    out_ref[...] = (
        jnp.dot(h, w2_ref[...], preferred_element_type=jnp.float32) + b2_ref[...]
    )


def kernel(input_ids, emb_table, W1, b1, W2, b2):
    B, S = input_ids.shape
    V, D = emb_table.shape
    H = W1.shape[1]
    C = W2.shape[1]

    idsT = jnp.transpose(input_ids.astype(jnp.int32))
    tabT = jnp.transpose(emb_table)
    pooled2 = _make_pooled_gather(V, D, B, S)(idsT, tabT)

    W1p = jnp.pad(W1, ((0, D), (0, 0)))
    W2p = jnp.pad(W2, ((0, 0), (0, _NCLS_PAD - C)))
    b2p = jnp.pad(b2, (0, _NCLS_PAD - C)).reshape(1, _NCLS_PAD)
    b1r = b1.reshape(1, H)

    BB = _MLP_BLOCK
    logits_pad = pl.pallas_call(
        _mlp_body,
        grid=(B // BB,),
        in_specs=[
            pl.BlockSpec((BB, 2 * D), lambda i: (i, 0)),
            pl.BlockSpec((2 * D, H), lambda i: (0, 0)),
            pl.BlockSpec((1, H), lambda i: (0, 0)),
            pl.BlockSpec((H, _NCLS_PAD), lambda i: (0, 0)),
            pl.BlockSpec((1, _NCLS_PAD), lambda i: (0, 0)),
        ],
        out_specs=pl.BlockSpec((BB, _NCLS_PAD), lambda i: (i, 0)),
        out_shape=jax.ShapeDtypeStruct((B, _NCLS_PAD), jnp.float32),
    )(pooled2, W1p, b1r, W2p, b2p)
    return logits_pad[:, :C]


# speedup vs baseline: 1.0493x; 1.0493x over previous
import functools

import jax
import jax.numpy as jnp
from jax import lax
from jax.experimental import pallas as pl
from jax.experimental.pallas import tpu as pltpu
from jax.experimental.pallas import tpu_sc as plsc

_NCLS_PAD = 128
_MLP_BLOCK = 2048
_LANES = 16
_NBUF = 8


@functools.lru_cache(maxsize=None)
def _make_pooled_gather(V, D, B, S):
    info = plsc.get_sparse_core_info()
    NC, NS = info.num_cores, info.num_subcores
    NW = NC * NS
    assert B % NW == 0 and (B // NW) % _LANES == 0
    b_per_w = B // NW
    row0 = ((S - 1) // 8) * 8
    sub = (S - 1) % 8
    mesh = plsc.VectorSubcoreMesh(core_axis_name="c", subcore_axis_name="s")

    @functools.partial(
        pl.kernel,
        mesh=mesh,
        compiler_params=pltpu.CompilerParams(needs_layout_passes=False),
        out_type=jax.ShapeDtypeStruct((B, 2 * D), jnp.float32),
        scratch_types=[
            pltpu.VMEM((8, b_per_w), jnp.int32),
            pltpu.VMEM((b_per_w + 2 * _LANES,), jnp.int32),
            pltpu.VMEM((b_per_w,), jnp.int32),
            pltpu.VMEM((_NBUF, D, 128), jnp.float32),
            pltpu.VMEM((b_per_w, 2 * D), jnp.float32),
        ] + [pltpu.SemaphoreType.DMA] * _NBUF,
    )
    def gather_last(idsT_hbm, tabT_hbm, out_hbm, ids_v, col_v, lane_v,
                    slabs_v, out_v, *sems):
        wid = lax.axis_index("s") * NC + lax.axis_index("c")
        base = wid * b_per_w
        pltpu.sync_copy(idsT_hbm.at[pl.ds(row0, 8), pl.ds(base, b_per_w)],
                        ids_v)
        for t in range(b_per_w // _LANES):
            ids16 = ids_v[sub, pl.ds(t * _LANES, _LANES)]
            col_v[pl.ds(t * _LANES, _LANES)] = lax.bitwise_and(ids16, ~127)
            lane_v[pl.ds(t * _LANES, _LANES)] = lax.bitwise_and(ids16, 127)

        head = col_v[pl.ds(0, _LANES)]
        for j in range(_NBUF):
            c0 = pl.multiple_of(head[j], 128)
            pltpu.async_copy(tabT_hbm.at[:, pl.ds(c0, 128)],
                             slabs_v.at[j], sems[j])

        def body(t, carry):
            col16 = col_v[pl.ds(t * _LANES, _LANES)]
            coln = col_v[pl.ds(t * _LANES + _LANES, _LANES)]
            lane16 = lane_v[pl.ds(t * _LANES, _LANES)]
            for j2 in range(_LANES):
                j = t * _LANES + j2
                slot = j2 % _NBUF
                pltpu.make_async_copy(tabT_hbm.at[:, pl.ds(0, 128)],
                                      slabs_v.at[slot], sems[slot]).wait()
                ln = jnp.broadcast_to(lane16[j2], (_LANES,))
                for k in range(D // _LANES):
                    e16 = k * _LANES + lax.iota(jnp.int32, _LANES)
                    v = plsc.load_gather(slabs_v.at[slot], [e16, ln])
                    out_v[j, pl.ds(k * _LANES, _LANES)] = v
                    out_v[j, pl.ds(D + k * _LANES, _LANES)] = v
                c0n = coln[j2 - _NBUF] if j2 >= _NBUF else col16[j2 + _NBUF]

                @pl.when(j + _NBUF < b_per_w)
                def _():
                    c0a = pl.multiple_of(c0n, 128)
                    pltpu.async_copy(tabT_hbm.at[:, pl.ds(c0a, 128)],
                                     slabs_v.at[slot], sems[slot])

            return carry

        lax.fori_loop(0, b_per_w // _LANES, body, 0)
        pltpu.sync_copy(out_v, out_hbm.at[pl.ds(base, b_per_w)])

    return gather_last


def _mlp_body(x_ref, w1_ref, b1_ref, w2_ref, b2_ref, out_ref):
    h = jnp.dot(x_ref[...], w1_ref[...], preferred_element_type=jnp.float32)
    h = jnp.maximum(h + b1_ref[...], 0.0)
    out_ref[...] = (
        jnp.dot(h, w2_ref[...], preferred_element_type=jnp.float32) + b2_ref[...]
    )


def kernel(input_ids, emb_table, W1, b1, W2, b2):
    B, S = input_ids.shape
    V, D = emb_table.shape
    H = W1.shape[1]
    C = W2.shape[1]

    idsT = jnp.transpose(input_ids.astype(jnp.int32))
    tabT = jnp.transpose(emb_table)
    pooled2 = _make_pooled_gather(V, D, B, S)(idsT, tabT)

    W1p = jnp.pad(W1, ((0, D), (0, 0)))
    W2p = jnp.pad(W2, ((0, 0), (0, _NCLS_PAD - C)))
    b2p = jnp.pad(b2, (0, _NCLS_PAD - C)).reshape(1, _NCLS_PAD)
    b1r = b1.reshape(1, H)

    BB = _MLP_BLOCK
    logits_pad = pl.pallas_call(
        _mlp_body,
        grid=(B // BB,),
        in_specs=[
            pl.BlockSpec((BB, 2 * D), lambda i: (i, 0)),
            pl.BlockSpec((2 * D, H), lambda i: (0, 0)),
            pl.BlockSpec((1, H), lambda i: (0, 0)),
            pl.BlockSpec((H, _NCLS_PAD), lambda i: (0, 0)),
            pl.BlockSpec((1, _NCLS_PAD), lambda i: (0, 0)),
        ],
        out_specs=pl.BlockSpec((BB, _NCLS_PAD), lambda i: (i, 0)),
        out_shape=jax.ShapeDtypeStruct((B, _NCLS_PAD), jnp.float32),
    )(pooled2, W1p, b1r, W2p, b2p)
    return logits_pad[:, :C]
